# baseline (speedup 1.0000x reference)
import functools

import jax
import jax.numpy as jnp
from jax.experimental import pallas as pl
from jax.experimental.pallas import tpu as pltpu
from jax.experimental.pallas import tpu_sc as plsc

H = 1024
E = 8
NREG = 5
NSH = 2
P2 = 16
LBW = 0.01

NB = 4096
BTS = 512
NTILES = 21
CAPS = NTILES * BTS
JTOT = 2 * NB
PERM = CAPS + JTOT
NC = 2
NS = 16
NW = NC * NS
SL = CAPS // NW
GCH = 48
JSC = JTOT // NS
JCW = JTOT // NW


def _router_body(x_ref, qf_ref, ln_g, ln_b, qw1, qb1, qw2, qb2, fgx, fgq, fgb,
                 wg1, wg2, cewg, cefw, cefb, cec, cpw, cpb,
                 xn_out, wd_out, part_out, rank_out, tki_out, tot_out,
                 disp_out, dens_out, cnt_run):
    f32 = jnp.float32
    t = pl.program_id(0)
    x = x_ref[...]
    mu = jnp.mean(x, axis=-1, keepdims=True)
    xc = x - mu
    var = jnp.mean(xc * xc, axis=-1, keepdims=True)
    xn = xc / jnp.sqrt(var + 1e-5) * ln_g[...] + ln_b[...]

    q = jnp.maximum(
        jnp.dot(qf_ref[...], qw1[...], preferred_element_type=f32) + qb1[...], 0.0)
    q = jnp.dot(q, qw2[...], preferred_element_type=f32) + qb2[...]
    fused = jnp.maximum(
        jnp.dot(xn, fgx[...], preferred_element_type=f32)
        + jnp.dot(q, fgq[...], preferred_element_type=f32) + fgb[...], 0.0)
    tl = jnp.tanh(jnp.dot(fused, wg1[...], preferred_element_type=f32))
    logits = jnp.dot(tl, wg2[...], preferred_element_type=f32)

    m = jnp.max(logits, axis=-1, keepdims=True)
    ez = jnp.exp(logits - m)
    ew = ez / jnp.sum(ez, axis=-1, keepdims=True)

    iota = jax.lax.broadcasted_iota(jnp.int32, ew.shape, 1)
    m1 = jnp.max(ew, axis=-1, keepdims=True)
    i1 = jnp.min(jnp.where(ew == m1, iota, E), axis=-1, keepdims=True)
    ewm = jnp.where(iota == i1, -1.0, ew)
    m2 = jnp.max(ewm, axis=-1, keepdims=True)
    i2 = jnp.min(jnp.where(ewm == m2, iota, E), axis=-1, keepdims=True)
    s = m1 + m2 + 1e-6
    wd = jnp.where(iota == i1, m1 / s, 0.0) + jnp.where(iota == i2, m2 / s, 0.0)

    z = jnp.dot(xn, cewg[...], preferred_element_type=f32)
    zm = jnp.max(z, axis=-1, keepdims=True)
    zez = jnp.exp(z - zm)
    cw = zez / jnp.sum(zez, axis=-1, keepdims=True)
    fc = jnp.dot(xn, cefw[...], preferred_element_type=f32) + cefb[...]
    ce_out = cw[:, 0:1] * fc + cw[:, 1:2] * cec[...]
    cp_out = jnp.dot(xn, cpw[...], preferred_element_type=f32) + cpb[...]
    part = wd[:, 5:6] * ce_out + wd[:, 6:7] * cp_out

    @pl.when(t == 0)
    def _():
        cnt_run[...] = jnp.zeros_like(cnt_run)

    bt = x.shape[0]
    ind = (wd > 0.0).astype(f32)
    ri = jax.lax.broadcasted_iota(jnp.int32, (bt, bt), 0)
    ci = jax.lax.broadcasted_iota(jnp.int32, (bt, bt), 1)
    tri = (ri > ci).astype(f32)
    rank = jnp.dot(tri, ind, preferred_element_type=f32) + cnt_run[...]
    tile_cnt = jnp.sum(ind, axis=0, keepdims=True)
    cnt_run[...] = cnt_run[...] + tile_cnt

    xn_out[...] = xn
    wd_out[...] = wd
    part_out[...] = part
    rank_out[...] = rank
    tki_out[...] = jnp.concatenate([i1, i2], axis=1)
    tot_out[...] = cnt_run[...].reshape(1, 1, E)
    disp_out[...] = tile_cnt.reshape(1, 1, E)
    dens_out[...] = jnp.sum(ew, axis=0, keepdims=True).reshape(1, 1, E)


def _meta_body(tot_ref, rank_ref, tki_ref, wd_ref,
               pos_s_out, pos_c_out, w2_out, te_out, gend_out, tval_out):
    f32 = jnp.float32
    i32 = jnp.int32
    cnt = tot_ref[0]
    lane = jax.lax.broadcasted_iota(i32, (1, E), 1)
    padded = jnp.where(lane < NREG, jnp.ceil(cnt / BTS) * BTS, 0.0)

    def shiftk(v, k):
        return jnp.concatenate([jnp.zeros((1, k), f32), v[:, :-k]], axis=1)

    inc = padded
    inc = inc + shiftk(inc, 1)
    inc = inc + shiftk(inc, 2)
    inc = inc + shiftk(inc, 4)
    base = inc - padded

    tiles = jax.lax.broadcasted_iota(i32, (1, 32), 1)
    tpos = (tiles * BTS).astype(f32)
    te = jnp.zeros((1, 32), i32)
    gend = jnp.zeros((1, 32), f32)
    for e in range(NREG):
        te = te + (tpos >= inc[0:1, e:e + 1]).astype(i32)
    te = jnp.minimum(te, NREG - 1)
    for e in range(NREG):
        gend = gend + jnp.where(te == e, base[0:1, e:e + 1] + cnt[0:1, e:e + 1], 0.0)
    tval = (tpos < gend).astype(i32)

    tokid = jax.lax.broadcasted_iota(i32, tki_ref.shape, 0)[:, 0:1]
    cols = []
    for k in range(2):
        trash = (CAPS + 2 * tokid + k).astype(f32)
        ek = tki_ref[:, k:k + 1]
        rk = jnp.zeros_like(rank_ref[:, 0:1])
        wk = jnp.zeros_like(rk)
        for e in range(NREG):
            sel = (ek == e)
            rk = rk + jnp.where(sel, rank_ref[:, e:e + 1] + base[0:1, e:e + 1], 0.0)
            wk = wk + jnp.where(sel, wd_ref[:, e:e + 1], 0.0)
        heavy = ek < NREG
        cols.append((jnp.where(heavy, rk, trash).astype(i32),
                     jnp.where(heavy, rk, 0.0).astype(i32), wk))
    pos_s_out[...] = jnp.concatenate([cols[0][0], cols[1][0]], axis=1)
    pos_c_out[...] = jnp.concatenate([cols[0][1], cols[1][1]], axis=1)
    w2_out[...] = jnp.concatenate([cols[0][2], cols[1][2]], axis=1)
    te_out[...] = te
    gend_out[...] = gend.astype(i32)
    tval_out[...] = tval


def _sc_mesh():
    return plsc.VectorSubcoreMesh(
        core_axis_name="c", subcore_axis_name="s",
        num_cores=NC, num_subcores=NS)


def _sc_dispatch_body(pos_hbm, tok_hbm, xn_hbm, xg_hbm, perm_sh, zbuf,
                      posv, tokv, permv, rows, sem):
    i32 = jnp.int32
    cid = jax.lax.axis_index("c")
    sid = jax.lax.axis_index("s")
    wid = sid * NC + cid
    zc = PERM // NS

    for i in range(zc // 16):
        zbuf[pl.ds(i * 16, 16)] = jnp.zeros((16,), i32)
    pltpu.sync_copy(zbuf, perm_sh.at[pl.ds(sid * zc, zc)])
    plsc.subcore_barrier()

    rbase = sid * (JSC // 128)
    pltpu.sync_copy(pos_hbm.at[pl.ds(rbase, JSC // 128)], posv)
    pltpu.sync_copy(tok_hbm.at[pl.ds(rbase, JSC // 128)], tokv)
    for r in range(JSC // 128):
        pltpu.sync_copy(tokv.at[r], perm_sh.at[posv.at[r]])
    plsc.subcore_barrier()

    sbase = wid * SL
    for g in range(SL // GCH):
        pltpu.sync_copy(perm_sh.at[pl.ds(sbase + g * GCH, GCH)], permv)
        pltpu.async_copy(xn_hbm.at[permv], rows, sem).wait()
        pltpu.sync_copy(rows, xg_hbm.at[pl.ds(sbase + g * GCH, GCH)])


def _sc_dispatch(pos2, tok2, xn):
    k = pl.kernel(
        _sc_dispatch_body, mesh=_sc_mesh(),
        out_type=jax.ShapeDtypeStruct((CAPS, H), jnp.float32),
        scratch_types=[
            pltpu.VMEM_SHARED((PERM,), jnp.int32),
            pltpu.VMEM((PERM // NS,), jnp.int32),
            pltpu.VMEM((JSC // 128, 128), jnp.int32),
            pltpu.VMEM((JSC // 128, 128), jnp.int32),
            pltpu.VMEM((GCH,), jnp.int32),
            pltpu.VMEM((GCH, H), jnp.float32),
            pltpu.SemaphoreType.DMA,
        ])
    return k(pos2, tok2, xn)


def _sc_combine_body(pos_hbm, slots_hbm, o2_hbm, posv, rows, sem):
    cid = jax.lax.axis_index("c")
    sid = jax.lax.axis_index("s")
    wid = sid * NC + cid
    jbase = wid * JCW
    for g in range(JCW // 128):
        pltpu.sync_copy(pos_hbm.at[pl.ds(jbase + g * 128, 128)], posv)
        pltpu.async_copy(slots_hbm.at[posv], rows, sem).wait()
        pltpu.sync_copy(rows, o2_hbm.at[pl.ds(jbase + g * 128, 128)])


def _sc_combine(pos_flat, out_slots):
    k = pl.kernel(
        _sc_combine_body, mesh=_sc_mesh(),
        out_type=jax.ShapeDtypeStruct((JTOT, 128), jnp.float32),
        scratch_types=[
            pltpu.VMEM((128,), jnp.int32),
            pltpu.VMEM((128, 128), jnp.float32),
            pltpu.SemaphoreType.DMA,
        ])
    return k(pos_flat, out_slots)


def _mlp(x, w1, b1, w2, b2, w3, b3):
    f32 = jnp.float32
    h = jnp.maximum(jnp.dot(x, w1, preferred_element_type=f32) + b1, 0.0)
    h = jnp.maximum(jnp.dot(h, w2, preferred_element_type=f32) + b2, 0.0)
    return jnp.dot(h, w3, preferred_element_type=f32) + b3


def _mlp_split(x, w1, b1, w2, b2, w3, b3):
    n = x.shape[0] // 2
    return jnp.concatenate(
        [_mlp(x[:n], w1, b1, w2, b2, w3, b3),
         _mlp(x[n:], w1, b1, w2, b2, w3, b3)], axis=0)


def _shared_body(xn_ref, w1a, b1a, w2a, b2a, w3a, b3a,
                 w1b, b1b, w2b, b2b, w3b, b3b, s0_out, s1_out):
    x = xn_ref[...]
    s0_out[...] = _mlp_split(x, w1a[...], b1a[...], w2a[...], b2a[...],
                             w3a[...], b3a[...])
    s1_out[...] = _mlp_split(x, w1b[...], b1b[...], w2b[...], b2b[...],
                             w3b[...], b3b[...])


def _routed_body(te_ref, gend_ref, tval_ref, xg_ref,
                 pw1_r, pb1_r, pw2_r, pb2_r, pw3_r, pb3_r, out_ref):
    t = pl.program_id(0)

    @pl.when(tval_ref[t] == 1)
    def _():
        out_ref[:, 0:P2] = _mlp_split(xg_ref[...], pw1_r[0], pb1_r[0],
                                      pw2_r[0], pb2_r[0], pw3_r[0], pb3_r[0])

    @pl.when(tval_ref[t] == 0)
    def _():
        out_ref[:, 0:P2] = jnp.zeros((out_ref.shape[0], P2), jnp.float32)


def _final_body(nt, nb, part_ref, o2_ref, w2_ref, s0_ref, s1_ref,
                opw0, opw1, opw2, opb, disp_ref, dens_ref,
                gauss_out, aux_out):
    f32 = jnp.float32
    t = pl.program_id(0)
    w0 = w2_ref[:, 0:1]
    w1 = w2_ref[:, 1:2]
    routed = (part_ref[...]
              + jnp.where(w0 > 0.0, w0 * o2_ref[:, 0:P2], 0.0)
              + jnp.where(w1 > 0.0, w1 * o2_ref[:, 128:128 + P2], 0.0))
    g = (jnp.dot(routed, opw0[...], preferred_element_type=f32)
         + jnp.dot(s0_ref[...], opw1[...], preferred_element_type=f32)
         + jnp.dot(s1_ref[...], opw2[...], preferred_element_type=f32)
         + opb[...])
    gauss_out[...] = g

    @pl.when(t == nt - 1)
    def _():
        cnt = jnp.sum(disp_ref[...], axis=0)
        dsum = jnp.sum(dens_ref[...], axis=0)
        val = (E * LBW) * jnp.sum(cnt * dsum) / (nb * nb)
        aux_out[...] = val.reshape(1, 1)


def kernel(multimodal_feat, query_feat, ln_g, ln_b, qe_w1, qe_b1, qe_w2, qe_b2,
           fg_w, fg_b, wg1, wg2, pw1, pb1, pw2, pb2, pw3, pb3, ce_const, ce_wg,
           ce_fc_w, ce_fc_b, cp_w, cp_b, sw1, sb1, sw2, sb2, sw3, sb3, op_w, op_b):
    f32 = jnp.float32
    i32 = jnp.int32
    nb = multimodal_feat.shape[0]
    bta = 1024
    btb = 1024
    na = nb // bta
    ntb = nb // btb

    r2 = lambda v: v.reshape(1, -1)
    fullspec = lambda a: pl.BlockSpec(
        a.shape, functools.partial(lambda nd, *_: (0,) * nd, a.ndim))

    ins_a = [
        multimodal_feat, query_feat, r2(ln_g), r2(ln_b),
        qe_w1, r2(qe_b1), qe_w2, r2(qe_b2),
        fg_w[:H], fg_w[H:], r2(fg_b), wg1, wg2,
        ce_wg, ce_fc_w, r2(ce_fc_b), r2(ce_const), cp_w, r2(cp_b),
    ]
    specs_a = [
        pl.BlockSpec((bta, H), lambda t: (t, 0)),
        pl.BlockSpec((bta, H), lambda t: (t, 0)),
    ] + [fullspec(a) for a in ins_a[2:]]
    out_shape_a = [
        jax.ShapeDtypeStruct((nb, H), f32),
        jax.ShapeDtypeStruct((nb, E), f32),
        jax.ShapeDtypeStruct((nb, P2), f32),
        jax.ShapeDtypeStruct((nb, E), f32),
        jax.ShapeDtypeStruct((nb, 2), i32),
        jax.ShapeDtypeStruct((1, 1, E), f32),
        jax.ShapeDtypeStruct((na, 1, E), f32),
        jax.ShapeDtypeStruct((na, 1, E), f32),
    ]
    out_specs_a = [
        pl.BlockSpec((bta, H), lambda t: (t, 0)),
        pl.BlockSpec((bta, E), lambda t: (t, 0)),
        pl.BlockSpec((bta, P2), lambda t: (t, 0)),
        pl.BlockSpec((bta, E), lambda t: (t, 0)),
        pl.BlockSpec((bta, 2), lambda t: (t, 0)),
        pl.BlockSpec((1, 1, E), lambda t: (0, 0, 0)),
        pl.BlockSpec((1, 1, E), lambda t: (t, 0, 0)),
        pl.BlockSpec((1, 1, E), lambda t: (t, 0, 0)),
    ]
    xn, wd, part, rank, tki, tot, disp, dens = pl.pallas_call(
        _router_body, grid=(na,), in_specs=specs_a,
        out_specs=out_specs_a, out_shape=out_shape_a,
        scratch_shapes=[pltpu.VMEM((1, E), f32)],
    )(*ins_a)

    pos_s, pos_c, w2c, te, gend, tval = pl.pallas_call(
        _meta_body, grid=(1,),
        in_specs=[fullspec(a) for a in (tot, rank, tki, wd)],
        out_specs=[
            pl.BlockSpec((nb, 2), lambda t: (0, 0)),
            pl.BlockSpec((nb, 2), lambda t: (0, 0)),
            pl.BlockSpec((nb, 2), lambda t: (0, 0)),
            pl.BlockSpec((1, 32), lambda t: (0, 0)),
            pl.BlockSpec((1, 32), lambda t: (0, 0)),
            pl.BlockSpec((1, 32), lambda t: (0, 0)),
        ],
        out_shape=[
            jax.ShapeDtypeStruct((nb, 2), i32),
            jax.ShapeDtypeStruct((nb, 2), i32),
            jax.ShapeDtypeStruct((nb, 2), f32),
            jax.ShapeDtypeStruct((1, 32), i32),
            jax.ShapeDtypeStruct((1, 32), i32),
            jax.ShapeDtypeStruct((1, 32), i32),
        ],
    )(tot, rank, tki, wd)

    tok2 = (jnp.arange(JTOT, dtype=i32) >> 1).reshape(JTOT // 128, 128)
    xg = _sc_dispatch(pos_s.reshape(JTOT // 128, 128), tok2, xn)

    ins_s = [
        xn,
        sw1[0], r2(sb1[0]), sw2[0], r2(sb2[0]), sw3[0], r2(sb3[0]),
        sw1[1], r2(sb1[1]), sw2[1], r2(sb2[1]), sw3[1], r2(sb3[1]),
    ]
    specs_s = [pl.BlockSpec((btb, H), lambda t: (t, 0))] + \
        [fullspec(a) for a in ins_s[1:]]
    s0, s1 = pl.pallas_call(
        _shared_body, grid=(ntb,), in_specs=specs_s,
        out_specs=[pl.BlockSpec((btb, P2), lambda t: (t, 0))] * 2,
        out_shape=[jax.ShapeDtypeStruct((nb, P2), f32)] * 2,
    )(*ins_s)

    te1 = te.reshape(-1)[:NTILES]
    gend1 = gend.reshape(-1)[:NTILES]
    tval1 = tval.reshape(-1)[:NTILES]

    grid_spec = pltpu.PrefetchScalarGridSpec(
        num_scalar_prefetch=3,
        grid=(NTILES,),
        in_specs=[
            pl.BlockSpec((BTS, H), lambda t, te_r, ge_r, tv_r: (t, 0)),
            pl.BlockSpec((1, H, 2 * H), lambda t, te_r, ge_r, tv_r: (te_r[t], 0, 0)),
            pl.BlockSpec((1, 1, 2 * H), lambda t, te_r, ge_r, tv_r: (te_r[t], 0, 0)),
            pl.BlockSpec((1, 2 * H, H), lambda t, te_r, ge_r, tv_r: (te_r[t], 0, 0)),
            pl.BlockSpec((1, 1, H), lambda t, te_r, ge_r, tv_r: (te_r[t], 0, 0)),
            pl.BlockSpec((1, H, P2), lambda t, te_r, ge_r, tv_r: (te_r[t], 0, 0)),
            pl.BlockSpec((1, 1, P2), lambda t, te_r, ge_r, tv_r: (te_r[t], 0, 0)),
        ],
        out_specs=pl.BlockSpec((BTS, 128), lambda t, te_r, ge_r, tv_r: (t, 0)),
    )
    out_slots = pl.pallas_call(
        _routed_body, grid_spec=grid_spec,
        out_shape=jax.ShapeDtypeStruct((CAPS, 128), f32),
    )(te1, gend1, tval1, xg,
      pw1, pb1.reshape(NREG, 1, 2 * H), pw2, pb2.reshape(NREG, 1, H),
      pw3, pb3.reshape(NREG, 1, P2))

    o2 = _sc_combine(pos_c.reshape(-1), out_slots)

    ins_f = [
        part, o2.reshape(nb, 256), w2c, s0, s1,
        op_w[0:P2], op_w[P2:2 * P2], op_w[2 * P2:], r2(op_b),
        disp, dens,
    ]
    specs_f = [
        pl.BlockSpec((btb, P2), lambda t: (t, 0)),
        pl.BlockSpec((btb, 256), lambda t: (t, 0)),
        pl.BlockSpec((btb, 2), lambda t: (t, 0)),
        pl.BlockSpec((btb, P2), lambda t: (t, 0)),
        pl.BlockSpec((btb, P2), lambda t: (t, 0)),
    ] + [fullspec(a) for a in ins_f[5:]]
    gauss, aux = pl.pallas_call(
        functools.partial(_final_body, ntb, float(nb)),
        grid=(ntb,), in_specs=specs_f,
        out_specs=[
            pl.BlockSpec((btb, P2), lambda t: (t, 0)),
            pl.BlockSpec((1, 1), lambda t: (0, 0)),
        ],
        out_shape=[
            jax.ShapeDtypeStruct((nb, P2), f32),
            jax.ShapeDtypeStruct((1, 1), f32),
        ],
    )(*ins_f)

    return gauss.reshape(-1, 2), aux[0, 0]


# speedup vs baseline: 1.6265x; 1.6265x over previous
import functools

import jax
import jax.numpy as jnp
from jax.experimental import pallas as pl
from jax.experimental.pallas import tpu as pltpu

H = 1024
E = 8
NREG = 5
NSH = 2
P2 = 16
LBW = 0.01


def _router_body(x_ref, qf_ref, ln_g, ln_b, qw1, qb1, qw2, qb2, fgx, fgq, fgb,
                 wg1, wg2, cewg, cefw, cefb, cec, cpw, cpb,
                 xn_out, wd_out, part_out, disp_out, dens_out):
    f32 = jnp.float32
    x = x_ref[...]
    mu = jnp.mean(x, axis=-1, keepdims=True)
    xc = x - mu
    var = jnp.mean(xc * xc, axis=-1, keepdims=True)
    xn = xc / jnp.sqrt(var + 1e-5) * ln_g[...] + ln_b[...]

    q = jnp.maximum(
        jnp.dot(qf_ref[...], qw1[...], preferred_element_type=f32) + qb1[...], 0.0)
    q = jnp.dot(q, qw2[...], preferred_element_type=f32) + qb2[...]
    fused = jnp.maximum(
        jnp.dot(xn, fgx[...], preferred_element_type=f32)
        + jnp.dot(q, fgq[...], preferred_element_type=f32) + fgb[...], 0.0)
    tl = jnp.tanh(jnp.dot(fused, wg1[...], preferred_element_type=f32))
    logits = jnp.dot(tl, wg2[...], preferred_element_type=f32)

    m = jnp.max(logits, axis=-1, keepdims=True)
    ez = jnp.exp(logits - m)
    ew = ez / jnp.sum(ez, axis=-1, keepdims=True)

    iota = jax.lax.broadcasted_iota(jnp.int32, ew.shape, 1)
    m1 = jnp.max(ew, axis=-1, keepdims=True)
    i1 = jnp.min(jnp.where(ew == m1, iota, E), axis=-1, keepdims=True)
    ewm = jnp.where(iota == i1, -1.0, ew)
    m2 = jnp.max(ewm, axis=-1, keepdims=True)
    i2 = jnp.min(jnp.where(ewm == m2, iota, E), axis=-1, keepdims=True)
    s = m1 + m2 + 1e-6
    wd = jnp.where(iota == i1, m1 / s, 0.0) + jnp.where(iota == i2, m2 / s, 0.0)

    z = jnp.dot(xn, cewg[...], preferred_element_type=f32)
    zm = jnp.max(z, axis=-1, keepdims=True)
    zez = jnp.exp(z - zm)
    cw = zez / jnp.sum(zez, axis=-1, keepdims=True)
    fc = jnp.dot(xn, cefw[...], preferred_element_type=f32) + cefb[...]
    ce_out = cw[:, 0:1] * fc + cw[:, 1:2] * cec[...]
    cp_out = jnp.dot(xn, cpw[...], preferred_element_type=f32) + cpb[...]
    part = wd[:, 5:6] * ce_out + wd[:, 6:7] * cp_out

    xn_out[...] = xn
    wd_out[...] = wd
    part_out[...] = part
    disp_out[...] = jnp.sum((wd > 0.0).astype(f32), axis=0, keepdims=True).reshape(1, 1, E)
    dens_out[...] = jnp.sum(ew, axis=0, keepdims=True).reshape(1, 1, E)


def _mlp(x, w1, b1, w2, b2, w3, b3):
    f32 = jnp.float32
    h = jnp.maximum(jnp.dot(x, w1, preferred_element_type=f32) + b1, 0.0)
    h = jnp.maximum(jnp.dot(h, w2, preferred_element_type=f32) + b2, 0.0)
    return jnp.dot(h, w3, preferred_element_type=f32) + b3


def _mlp_split(x, w1, b1, w2, b2, w3, b3):
    n = x.shape[0] // 2
    return jnp.concatenate(
        [_mlp(x[:n], w1, b1, w2, b2, w3, b3),
         _mlp(x[n:], w1, b1, w2, b2, w3, b3)], axis=0)


def _shared_body(xn_ref, w1a, b1a, w2a, b2a, w3a, b3a,
                 w1b, b1b, w2b, b2b, w3b, b3b, s0_out, s1_out):
    x = xn_ref[...]
    s0_out[...] = _mlp_split(x, w1a[...], b1a[...], w2a[...], b2a[...],
                             w3a[...], b3a[...])
    s1_out[...] = _mlp_split(x, w1b[...], b1b[...], w2b[...], b2b[...],
                             w3b[...], b3b[...])


def _routed_body(nt, nb, xn_ref, wd_ref, part_ref, s0_ref, s1_ref,
                 pw1_r, pb1_r, pw2_r, pb2_r, pw3_r, pb3_r,
                 opw0, opw1, opw2, opb, disp_ref, dens_ref,
                 gauss_out, aux_out, acc_ref):
    f32 = jnp.float32
    t = pl.program_id(0)
    e = pl.program_id(1)
    x = xn_ref[...]
    out = _mlp_split(x, pw1_r[0], pb1_r[0], pw2_r[0], pb2_r[0],
                     pw3_r[0], pb3_r[0])

    lane = jax.lax.broadcasted_iota(jnp.int32, (x.shape[0], E), 1)
    w_col = jnp.sum(jnp.where(lane == e, wd_ref[...], 0.0), axis=-1, keepdims=True)

    @pl.when(e == 0)
    def _():
        acc_ref[...] = part_ref[...] + w_col * out

    @pl.when(e > 0)
    def _():
        acc_ref[...] = acc_ref[...] + w_col * out

    @pl.when(e == NREG - 1)
    def _():
        g = (jnp.dot(acc_ref[...], opw0[...], preferred_element_type=f32)
             + jnp.dot(s0_ref[...], opw1[...], preferred_element_type=f32)
             + jnp.dot(s1_ref[...], opw2[...], preferred_element_type=f32)
             + opb[...])
        gauss_out[...] = g

    @pl.when(jnp.logical_and(t == nt - 1, e == NREG - 1))
    def _():
        cnt = jnp.sum(disp_ref[...], axis=0)
        dsum = jnp.sum(dens_ref[...], axis=0)
        val = (E * LBW) * jnp.sum(cnt * dsum) / (nb * nb)
        aux_out[...] = val.reshape(1, 1)


def kernel(multimodal_feat, query_feat, ln_g, ln_b, qe_w1, qe_b1, qe_w2, qe_b2,
           fg_w, fg_b, wg1, wg2, pw1, pb1, pw2, pb2, pw3, pb3, ce_const, ce_wg,
           ce_fc_w, ce_fc_b, cp_w, cp_b, sw1, sb1, sw2, sb2, sw3, sb3, op_w, op_b):
    f32 = jnp.float32
    nb = multimodal_feat.shape[0]
    bta = 1024
    btb = 1024
    na = nb // bta
    ntb = nb // btb

    r2 = lambda v: v.reshape(1, -1)
    fullspec = lambda a, ng: pl.BlockSpec(
        a.shape, functools.partial(lambda nd, *_: (0,) * nd, a.ndim))

    ins_a = [
        multimodal_feat, query_feat, r2(ln_g), r2(ln_b),
        qe_w1, r2(qe_b1), qe_w2, r2(qe_b2),
        fg_w[:H], fg_w[H:], r2(fg_b), wg1, wg2,
        ce_wg, ce_fc_w, r2(ce_fc_b), r2(ce_const), cp_w, r2(cp_b),
    ]
    specs_a = [
        pl.BlockSpec((bta, H), lambda t: (t, 0)),
        pl.BlockSpec((bta, H), lambda t: (t, 0)),
    ] + [fullspec(a, 1) for a in ins_a[2:]]
    out_shape_a = [
        jax.ShapeDtypeStruct((nb, H), f32),
        jax.ShapeDtypeStruct((nb, E), f32),
        jax.ShapeDtypeStruct((nb, P2), f32),
        jax.ShapeDtypeStruct((na, 1, E), f32),
        jax.ShapeDtypeStruct((na, 1, E), f32),
    ]
    out_specs_a = [
        pl.BlockSpec((bta, H), lambda t: (t, 0)),
        pl.BlockSpec((bta, E), lambda t: (t, 0)),
        pl.BlockSpec((bta, P2), lambda t: (t, 0)),
        pl.BlockSpec((1, 1, E), lambda t: (t, 0, 0)),
        pl.BlockSpec((1, 1, E), lambda t: (t, 0, 0)),
    ]
    xn, wd, part, disp, dens = pl.pallas_call(
        _router_body, grid=(na,), in_specs=specs_a,
        out_specs=out_specs_a, out_shape=out_shape_a,
    )(*ins_a)

    ins_s = [
        xn,
        sw1[0], r2(sb1[0]), sw2[0], r2(sb2[0]), sw3[0], r2(sb3[0]),
        sw1[1], r2(sb1[1]), sw2[1], r2(sb2[1]), sw3[1], r2(sb3[1]),
    ]
    specs_s = [pl.BlockSpec((btb, H), lambda t: (t, 0))] + \
        [fullspec(a, 1) for a in ins_s[1:]]
    s0, s1 = pl.pallas_call(
        _shared_body, grid=(ntb,), in_specs=specs_s,
        out_specs=[pl.BlockSpec((btb, P2), lambda t: (t, 0))] * 2,
        out_shape=[jax.ShapeDtypeStruct((nb, P2), f32)] * 2,
    )(*ins_s)

    ins_r = [
        xn, wd, part, s0, s1,
        pw1, pb1.reshape(NREG, 1, 2 * H), pw2, pb2.reshape(NREG, 1, H),
        pw3, pb3.reshape(NREG, 1, P2),
        op_w[0:P2], op_w[P2:2 * P2], op_w[2 * P2:], r2(op_b),
        disp, dens,
    ]
    specs_r = [
        pl.BlockSpec((btb, H), lambda t, e: (t, 0)),
        pl.BlockSpec((btb, E), lambda t, e: (t, 0)),
        pl.BlockSpec((btb, P2), lambda t, e: (t, 0)),
        pl.BlockSpec((btb, P2), lambda t, e: (t, 0)),
        pl.BlockSpec((btb, P2), lambda t, e: (t, 0)),
        pl.BlockSpec((1, H, 2 * H), lambda t, e: (e, 0, 0)),
        pl.BlockSpec((1, 1, 2 * H), lambda t, e: (e, 0, 0)),
        pl.BlockSpec((1, 2 * H, H), lambda t, e: (e, 0, 0)),
        pl.BlockSpec((1, 1, H), lambda t, e: (e, 0, 0)),
        pl.BlockSpec((1, H, P2), lambda t, e: (e, 0, 0)),
        pl.BlockSpec((1, 1, P2), lambda t, e: (e, 0, 0)),
        pl.BlockSpec((P2, P2), lambda t, e: (0, 0)),
        pl.BlockSpec((P2, P2), lambda t, e: (0, 0)),
        pl.BlockSpec((P2, P2), lambda t, e: (0, 0)),
        pl.BlockSpec((1, P2), lambda t, e: (0, 0)),
        pl.BlockSpec((na, 1, E), lambda t, e: (0, 0, 0)),
        pl.BlockSpec((na, 1, E), lambda t, e: (0, 0, 0)),
    ]
    gauss, aux = pl.pallas_call(
        functools.partial(_routed_body, ntb, float(nb)),
        grid=(ntb, NREG),
        in_specs=specs_r,
        out_specs=[
            pl.BlockSpec((btb, P2), lambda t, e: (t, 0)),
            pl.BlockSpec((1, 1), lambda t, e: (0, 0)),
        ],
        out_shape=[
            jax.ShapeDtypeStruct((nb, P2), f32),
            jax.ShapeDtypeStruct((1, 1), f32),
        ],
        scratch_shapes=[pltpu.VMEM((btb, P2), f32)],
    )(*ins_r)

    return gauss.reshape(-1, 2), aux[0, 0]
